# baseline (speedup 1.0000x reference)
import functools

import jax
import jax.numpy as jnp
from jax import lax
from jax.experimental import pallas as pl
from jax.experimental.pallas import tpu as pltpu
from jax.experimental.pallas import tpu_sc as plsc

E = 64
TOP_K = 8
N_GROUP = 8
TOPK_GROUP = 3
GROUP_SIZE = E // N_GROUP
SCALE = 16.0

BT = 512
NW = 32
L = 16


def _scores_kernel(x_ref, w_ref, s_ref):
    x = x_ref[...]
    w = w_ref[...]
    logits = jax.lax.dot_general(
        x, w, (((1,), (1,)), ((), ())),
        preferred_element_type=jnp.float32,
    )
    m = jnp.max(logits, axis=1, keepdims=True)
    ex = jnp.exp(logits - m)
    s_ref[...] = ex / jnp.sum(ex, axis=1, keepdims=True)


def _tc_scores(x, w, n_tokens, hidden_dim):
    return pl.pallas_call(
        _scores_kernel,
        grid=(n_tokens // BT,),
        in_specs=[
            pl.BlockSpec((BT, hidden_dim), lambda i: (i, 0)),
            pl.BlockSpec((E, hidden_dim), lambda i: (0, 0)),
        ],
        out_specs=pl.BlockSpec((BT, E), lambda i: (i, 0)),
        out_shape=jax.ShapeDtypeStruct((n_tokens, E), jnp.float32),
    )(x, w)


def _sc_route_body(scores_hbm, idx_hbm, wgt_hbm, sv, iv, wv):
    t_per_w = sv.shape[0] // E
    iota = lax.iota(jnp.int32, L)

    gd = lax.GatherDimensionNumbers(
        offset_dims=(), collapsed_slice_dims=(0,), start_index_map=(0,))

    def take(a, idx):
        return lax.gather(a, idx[:, None], gd, slice_sizes=(1,),
                          mode=lax.GatherScatterMode.PROMISE_IN_BOUNDS)

    wid = lax.axis_index("s") * 2 + lax.axis_index("c")
    pltpu.sync_copy(scores_hbm.at[pl.ds(wid * sv.shape[0], sv.shape[0])], sv)

    perms = [iota ^ 1, iota ^ 2, iota ^ 4]
    zeros = jnp.zeros((L,), jnp.int32)
    ones = zeros + 1
    twos = zeros + 2
    eights = zeros + 8
    shift8 = (iota - 8) & 15
    lane_lt8 = iota < 8
    idvecs = [iota + 16 * r for r in range(4)]
    grpvecs = [(iota + 16 * r) >> 3 for r in range(4)]

    def route_one(tok):
        v = [sv[pl.ds(tok * E + r * L, L)] for r in range(4)]
        m = v
        for p in perms:
            m = [jnp.maximum(mr, take(mr, p)) for mr in m]
        g = jnp.full((L,), -1.0, jnp.float32)
        for r in range(4):
            g = jnp.where(iota == 2 * r, take(m[r], zeros), g)
            g = jnp.where(iota == 2 * r + 1, take(m[r], eights), g)
        gk, gv_ids = plsc.sort_key_val(g, iota, descending=True)
        b0 = take(gv_ids, zeros)
        b1 = take(gv_ids, ones)
        b2 = take(gv_ids, twos)
        ks, xs = [], []
        for r in range(4):
            sel = (grpvecs[r] == b0) | (grpvecs[r] == b1) | (grpvecs[r] == b2)
            wr = jnp.where(sel, v[r], -1.0)
            kr, xr = plsc.sort_key_val(wr, idvecs[r], descending=True)
            ks.append(kr)
            xs.append(xr)

        def merge(ka, va, kb, vb):
            ck = jnp.where(lane_lt8, ka, lax.rev(kb, (0,)))
            cv = jnp.where(lane_lt8, va, lax.rev(vb, (0,)))
            return plsc.sort_key_val(ck, cv, descending=True)

        k01, v01 = merge(ks[0], xs[0], ks[1], xs[1])
        k23, v23 = merge(ks[2], xs[2], ks[3], xs[3])
        kf, vf = merge(k01, v01, k23, v23)
        return kf, vf

    def body(t, _):
        ka, va = route_one(2 * t)
        kb, vb = route_one(2 * t + 1)
        iv[pl.ds(t * L, L)] = jnp.where(lane_lt8, va, take(vb, shift8))
        wv[pl.ds(t * L, L)] = jnp.where(lane_lt8, ka, take(kb, shift8)) * SCALE
        return 0

    lax.fori_loop(0, t_per_w // 2, body, 0)

    pltpu.sync_copy(iv, idx_hbm.at[pl.ds(wid * iv.shape[0], iv.shape[0])])
    pltpu.sync_copy(wv, wgt_hbm.at[pl.ds(wid * wv.shape[0], wv.shape[0])])


def _sc_route(scores, n_tokens):
    t_per_w = n_tokens // NW
    mesh = plsc.VectorSubcoreMesh(core_axis_name="c", subcore_axis_name="s")
    run = pl.kernel(
        _sc_route_body,
        out_type=[
            jax.ShapeDtypeStruct((n_tokens * TOP_K,), jnp.int32),
            jax.ShapeDtypeStruct((n_tokens * TOP_K,), jnp.float32),
        ],
        mesh=mesh,
        compiler_params=pltpu.CompilerParams(needs_layout_passes=False),
        scratch_types=[
            pltpu.VMEM((t_per_w * E,), jnp.float32),
            pltpu.VMEM((t_per_w * TOP_K,), jnp.int32),
            pltpu.VMEM((t_per_w * TOP_K,), jnp.float32),
        ],
    )
    idx1, wgt1 = run(scores.reshape(n_tokens * E))
    return (idx1.reshape(n_tokens, TOP_K), wgt1.reshape(n_tokens, TOP_K))


def kernel(hidden_states, weight):
    bsz, seq_len, hidden_dim = hidden_states.shape
    n_tokens = bsz * seq_len
    x = hidden_states.reshape(n_tokens, hidden_dim).astype(jnp.float32)
    w = weight.astype(jnp.float32)
    scores = _tc_scores(x, w, n_tokens, hidden_dim)
    return _sc_route(scores, n_tokens)


# speedup vs baseline: 1.5673x; 1.5673x over previous
import jax
import jax.numpy as jnp
from jax.experimental import pallas as pl

E = 64
TOP_K = 8
N_GROUP = 8
TOPK_GROUP = 3
GROUP_SIZE = E // N_GROUP
SCALE = 16.0

BT = 1024


def _gate_kernel(x_ref, w_ref, idx_ref, wgt_ref):
    x = x_ref[...]
    w = w_ref[...]
    logits = jax.lax.dot_general(
        x, w, (((1,), (1,)), ((), ())),
        preferred_element_type=jnp.float32,
    )

    lt = logits.T
    m = jnp.max(lt, axis=0, keepdims=True)
    ex = jnp.exp(lt - m)
    scores = ex / jnp.sum(ex, axis=0, keepdims=True)

    bt = scores.shape[1]
    gsf = jnp.max(scores.reshape(N_GROUP, GROUP_SIZE, bt), axis=1)

    giota = jax.lax.broadcasted_iota(jnp.int32, (N_GROUP, bt), 0)
    gmask = jnp.zeros((N_GROUP, bt), jnp.bool_)
    gwork = gsf
    for _ in range(TOPK_GROUP):
        gm = jnp.max(gwork, axis=0, keepdims=True)
        eq = gwork == gm
        first = jnp.min(jnp.where(eq, giota, N_GROUP), axis=0, keepdims=True)
        sel = giota == first
        gmask = jnp.logical_or(gmask, sel)
        gwork = jnp.where(sel, -1.0, gwork)

    emask = jnp.broadcast_to(
        gmask.reshape(N_GROUP, 1, bt), (N_GROUP, GROUP_SIZE, bt)
    ).reshape(E, bt)
    tmp = jnp.where(emask, scores, -1.0)

    eiota = jax.lax.broadcasted_iota(jnp.int32, (E, bt), 0)
    idx_rows, wgt_rows = [], []
    for _ in range(TOP_K):
        km = jnp.max(tmp, axis=0, keepdims=True)
        eq = tmp == km
        first = jnp.min(jnp.where(eq, eiota, E), axis=0, keepdims=True)
        idx_rows.append(first)
        wgt_rows.append(km)
        tmp = jnp.where(eiota == first, -1.0, tmp)

    idx_ref[...] = jnp.concatenate(idx_rows, axis=0).T
    wgt_ref[...] = (jnp.concatenate(wgt_rows, axis=0) * SCALE).T


def kernel(hidden_states, weight):
    bsz, seq_len, hidden_dim = hidden_states.shape
    n_tokens = bsz * seq_len
    x = hidden_states.reshape(n_tokens, hidden_dim).astype(jnp.float32)
    w = weight.astype(jnp.float32)

    grid = (n_tokens // BT,)
    idx_t, wgt_t = pl.pallas_call(
        _gate_kernel,
        grid=grid,
        in_specs=[
            pl.BlockSpec((BT, hidden_dim), lambda i: (i, 0)),
            pl.BlockSpec((E, hidden_dim), lambda i: (0, 0)),
        ],
        out_specs=[
            pl.BlockSpec((BT, TOP_K), lambda i: (i, 0)),
            pl.BlockSpec((BT, TOP_K), lambda i: (i, 0)),
        ],
        out_shape=[
            jax.ShapeDtypeStruct((n_tokens, TOP_K), jnp.int32),
            jax.ShapeDtypeStruct((n_tokens, TOP_K), jnp.float32),
        ],
    )(x, w)
    return idx_t, wgt_t


# speedup vs baseline: 1.8585x; 1.1858x over previous
import jax
import jax.numpy as jnp
from jax.experimental import pallas as pl

E = 64
TOP_K = 8
N_GROUP = 8
TOPK_GROUP = 3
GROUP_SIZE = E // N_GROUP
SCALE = 16.0

BT = 1024


def _gate_kernel(x_ref, w_ref, idx_ref, wgt_ref):
    x = x_ref[...]
    w = w_ref[...]
    logits = jax.lax.dot_general(
        x, w, (((1,), (1,)), ((), ())),
        preferred_element_type=jnp.float32,
    )

    lt = logits.T
    m = jnp.max(lt, axis=0, keepdims=True)
    ex = jnp.exp(lt - m)
    scores = ex / jnp.sum(ex, axis=0, keepdims=True)

    bt = scores.shape[1]
    gsf = jnp.max(scores.reshape(N_GROUP, GROUP_SIZE, bt), axis=1)

    giota = jax.lax.broadcasted_iota(jnp.int32, (N_GROUP, bt), 0)
    gmask = jnp.zeros((N_GROUP, bt), jnp.bool_)
    gwork = gsf
    for _ in range(TOPK_GROUP):
        gm = jnp.max(gwork, axis=0, keepdims=True)
        eq = gwork == gm
        first = jnp.min(jnp.where(eq, giota, N_GROUP), axis=0, keepdims=True)
        sel = giota == first
        gmask = jnp.logical_or(gmask, sel)
        gwork = jnp.where(sel, -1.0, gwork)

    emask = jnp.broadcast_to(
        gmask.reshape(N_GROUP, 1, bt), (N_GROUP, GROUP_SIZE, bt)
    ).reshape(E, bt)
    tmp = jnp.where(emask, scores, -1.0)

    eiota = jax.lax.broadcasted_iota(jnp.int32, (E, bt), 0)
    idx_rows, wgt_rows = [], []
    for _ in range(TOP_K):
        km = jnp.max(tmp, axis=0, keepdims=True)
        eq = tmp == km
        first = jnp.min(jnp.where(eq, eiota, E), axis=0, keepdims=True)
        idx_rows.append(first)
        wgt_rows.append(km)
        tmp = jnp.where(eiota == first, -1.0, tmp)

    idx_ref[...] = jnp.concatenate(idx_rows, axis=0)
    wgt_ref[...] = jnp.concatenate(wgt_rows, axis=0) * SCALE


def kernel(hidden_states, weight):
    bsz, seq_len, hidden_dim = hidden_states.shape
    n_tokens = bsz * seq_len
    x = hidden_states.reshape(n_tokens, hidden_dim).astype(jnp.float32)
    w = weight.astype(jnp.float32)

    grid = (n_tokens // BT,)
    idx_t, wgt_t = pl.pallas_call(
        _gate_kernel,
        grid=grid,
        in_specs=[
            pl.BlockSpec((BT, hidden_dim), lambda i: (i, 0)),
            pl.BlockSpec((E, hidden_dim), lambda i: (0, 0)),
        ],
        out_specs=[
            pl.BlockSpec((TOP_K, BT), lambda i: (0, i)),
            pl.BlockSpec((TOP_K, BT), lambda i: (0, i)),
        ],
        out_shape=[
            jax.ShapeDtypeStruct((TOP_K, n_tokens), jnp.int32),
            jax.ShapeDtypeStruct((TOP_K, n_tokens), jnp.float32),
        ],
    )(x, w)
    return idx_t.T, wgt_t.T
